# Initial kernel scaffold; baseline (speedup 1.0000x reference)
import jax
import jax.numpy as jnp
from jax.experimental import pallas as pl


def kernel(x, stoken_size):
    raise NotImplementedError("write your pallas kernel here")


# speedup vs baseline: 2499.2953x; 2499.2953x over previous
import jax
import jax.numpy as jnp
from jax import lax
from jax.experimental import pallas as pl
from jax.experimental.pallas import tpu as pltpu

_H = 224
_W = 224
_SH = 16
_NH = 14
_S = _NH * _NH
_C = 96
_N = _H * _W
_BAND = _SH * _W
_NEG = -1e30
_TN = (((0,), (0,)), ((), ()))
_NT = (((1,), (1,)), ((), ()))


def _body(x_ref, out_ref, centT, cnorm, acc, cnt, meansT):
    ph = pl.program_id(0)
    c = pl.program_id(1)

    ri = lax.broadcasted_iota(jnp.int32, (_BAND, _S), 0)
    ji = lax.broadcasted_iota(jnp.int32, (_BAND, _S), 1)
    rf = ri.astype(jnp.float32)
    jf = ji.astype(jnp.float32)
    rb = jnp.floor(rf * (1.0 / 16.0))
    bx = rb - 14.0 * jnp.floor((rb + 0.5) * (1.0 / 14.0))
    sy = jnp.floor((jf + 0.5) * (1.0 / 14.0))
    sx = jf - 14.0 * sy
    cf = c.astype(jnp.float32)

    @pl.when((c == 0) & (ph == 1))
    def _():
        cn = acc[...]
        centT[...] = cn
        cnorm[...] = jnp.sum(cn * cn, axis=0, keepdims=True)

    @pl.when((c == 0) & ((ph == 2) | (ph == 3)))
    def _():
        cn = acc[...] / (cnt[...] + 1e-16)
        centT[...] = cn
        cnorm[...] = jnp.sum(cn * cn, axis=0, keepdims=True)

    @pl.when((c == 0) & (ph == 4))
    def _():
        meansT[...] = acc[...] / jnp.maximum(cnt[...], 1.0)

    @pl.when((c == 0) & (ph < 4))
    def _():
        acc[...] = jnp.zeros_like(acc)
        cnt[...] = jnp.zeros_like(cnt)

    X = x_ref[pl.ds(c * _BAND, _BAND), :]

    @pl.when(ph == 0)
    def _():
        w = jnp.where((sy == cf) & (sx == bx), 1.0 / 256.0, 0.0)
        acc[...] += lax.dot_general(X, w, _TN,
                                    preferred_element_type=jnp.float32, precision=lax.Precision.HIGHEST)

    valid = (jnp.abs(sy - cf) <= 1.0) & (jnp.abs(sx - bx) <= 1.0)

    @pl.when((ph == 1) | (ph == 2))
    def _():
        lm = 2.0 * jnp.dot(X, centT[...],
                           preferred_element_type=jnp.float32, precision=lax.Precision.HIGHEST) - cnorm[...]
        lm = jnp.where(valid, lm, _NEG)
        m = jnp.max(lm, axis=1, keepdims=True)
        e = jnp.exp(lm - m)
        aff = e / jnp.sum(e, axis=1, keepdims=True)
        acc[...] += lax.dot_general(X, aff, _TN,
                                    preferred_element_type=jnp.float32)
        cnt[...] += jnp.sum(aff, axis=0, keepdims=True)

    @pl.when((ph == 3) | (ph == 4))
    def _():
        lm = 2.0 * jnp.dot(X, centT[...],
                           preferred_element_type=jnp.float32, precision=lax.Precision.HIGHEST) - cnorm[...]
        lm = jnp.where(valid, lm, _NEG)
        m = jnp.max(lm, axis=1, keepdims=True)
        candj = jnp.where(lm >= m, ji, _S)
        labj = jnp.min(candj, axis=1, keepdims=True)
        onehot = (ji == labj).astype(jnp.float32)

        @pl.when(ph == 3)
        def _():
            acc[...] += lax.dot_general(X, onehot, _TN,
                                        preferred_element_type=jnp.float32, precision=lax.Precision.HIGHEST)
            cnt[...] += jnp.sum(onehot, axis=0, keepdims=True)

        @pl.when(ph == 4)
        def _():
            out_ref[...] = lax.dot_general(onehot, meansT[...], _NT,
                                           preferred_element_type=jnp.float32, precision=lax.Precision.HIGHEST)


def _ssn_pallas(x_pix):
    return pl.pallas_call(
        _body,
        grid=(5, _NH),
        in_specs=[pl.BlockSpec((_N, _C), lambda ph, c: (0, 0))],
        out_specs=pl.BlockSpec((_BAND, _C), lambda ph, c: (c, 0)),
        out_shape=jax.ShapeDtypeStruct((_N, _C), jnp.float32),
        scratch_shapes=[
            pltpu.VMEM((_C, _S), jnp.float32),
            pltpu.VMEM((1, _S), jnp.float32),
            pltpu.VMEM((_C, _S), jnp.float32),
            pltpu.VMEM((1, _S), jnp.float32),
            pltpu.VMEM((_C, _S), jnp.float32),
        ],
        compiler_params=pltpu.CompilerParams(
            dimension_semantics=("arbitrary", "arbitrary"),
            vmem_limit_bytes=100 * 1024 * 1024,
        ),
    )(x_pix)


def kernel(x, stoken_size):
    del stoken_size
    x_pix = x[0].reshape(_C, _N).T
    out_pix = _ssn_pallas(x_pix)
    return out_pix.T.reshape(1, _C, _H, _W)
